# baseline (speedup 1.0000x reference)
import jax
import jax.numpy as jnp
from jax import lax
from jax.experimental import pallas as pl
from jax.experimental.pallas import tpu as pltpu
from jax.experimental.pallas import tpu_sc as plsc

EMBED_DIM = 64
SCALE = 8.0
L = 16
NC, NS = 2, 16
NW = NC * NS
CHUNK = 128
GSIZE = 128
NSLOT = 10
PREF = NSLOT - 2
NGROUP = CHUNK // GSIZE


_GATHER_DN = lax.GatherDimensionNumbers(
    offset_dims=(), collapsed_slice_dims=(0,), start_index_map=(0,))


def _splat_lane(y, k):
    idx = jnp.full((L, 1), k, jnp.int32)
    return lax.gather(y, idx, _GATHER_DN, (1,),
                      mode=lax.GatherScatterMode.PROMISE_IN_BOUNDS)


def _normalize_rows(rows_v, chunk):
    lane = lax.iota(jnp.int32, L)

    def body16(i, carry):
        r0 = i * 16
        for q in range(4):
            base = r0 + 4 * q
            acc = jnp.full((L,), 1.0, jnp.float32)
            vs = []
            for k in range(4):
                r = base + k
                v0 = rows_v[r, pl.ds(0, L)]
                v1 = rows_v[r, pl.ds(L, L)]
                v2 = rows_v[r, pl.ds(2 * L, L)]
                v3 = rows_v[r, pl.ds(3 * L, L)]
                s = jnp.sum((v0 * v0 + v1 * v1) + (v2 * v2 + v3 * v3))
                acc = jnp.where(lane == k, jnp.full((L,), s, jnp.float32), acc)
                vs.append((r, v0, v1, v2, v3))
            sv = jnp.maximum(acc, 1e-24)
            iv = plsc.bitcast(sv, jnp.int32)
            y = plsc.bitcast(
                jnp.full((L,), 0x5F3759DF, jnp.int32) - (iv >> 1), jnp.float32)
            for _ in range(3):
                y = y * (1.5 - 0.5 * sv * y * y)
            y = y * SCALE
            for k, (r, v0, v1, v2, v3) in enumerate(vs):
                sc = _splat_lane(y, k)
                rows_v[r, pl.ds(0, L)] = v0 * sc
                rows_v[r, pl.ds(L, L)] = v1 * sc
                rows_v[r, pl.ds(2 * L, L)] = v2 * sc
                rows_v[r, pl.ds(3 * L, L)] = v3 * sc
        return carry

    lax.fori_loop(0, chunk // 16, body16, 0)


def _sc_body(idx_hbm, table_hbm, out_hbm, idx_v, rows, gsems, osems, b_per_w):
    wid = lax.axis_index("s") * NC + lax.axis_index("c")
    base = wid * b_per_w
    nchunks = b_per_w // CHUNK

    pltpu.sync_copy(idx_hbm.at[pl.ds(base, b_per_w)], idx_v)

    def fire_gather(g, slot):
        for j in range(NGROUP):
            pltpu.async_copy(
                table_hbm.at[idx_v.at[pl.ds(g * CHUNK + j * GSIZE, GSIZE)]],
                rows[slot].at[pl.ds(j * GSIZE, GSIZE)],
                gsems[slot])

    def wait_gather(g, slot):
        for j in range(NGROUP):
            pltpu.make_async_copy(
                table_hbm.at[idx_v.at[pl.ds(g * CHUNK + j * GSIZE, GSIZE)]],
                rows[slot].at[pl.ds(j * GSIZE, GSIZE)],
                gsems[slot]).wait()

    def fire_out(g, slot):
        pltpu.async_copy(
            rows[slot], out_hbm.at[pl.ds(base + g * CHUNK, CHUNK)], osems[slot])

    def wait_out(g, slot):
        pltpu.make_async_copy(
            rows[slot], out_hbm.at[pl.ds(base + g * CHUNK, CHUNK)],
            osems[slot]).wait()

    for g in range(PREF):
        fire_gather(g, g)

    def ring_body(gq, carry):
        for b in range(NSLOT):
            g = gq * NSLOT + b
            nxt = (b + PREF) % NSLOT

            @pl.when(g >= 2)
            def _():
                wait_out(g - 2, nxt)

            @pl.when(g + PREF < nchunks)
            def _():
                fire_gather(g + PREF, nxt)

            wait_gather(g, b)
            _normalize_rows(rows[b], CHUNK)
            fire_out(g, b)
        return carry

    lax.fori_loop(0, nchunks // NSLOT, ring_body, 0)

    for g in (nchunks - 2, nchunks - 1):
        wait_out(g, g % NSLOT)


def kernel(x, embed_mat):
    b0, seq = x.shape
    b = b0 * seq
    b_per_w = b // NW
    assert b % NW == 0 and b_per_w % (NSLOT * CHUNK) == 0
    idx = x.reshape(b).astype(jnp.int32)
    mesh = plsc.VectorSubcoreMesh(core_axis_name="c", subcore_axis_name="s")

    def body(idx_h, tab_h, out_h, idx_v, *rest):
        _sc_body(idx_h, tab_h, out_h, idx_v,
                 list(rest[:NSLOT]),
                 list(rest[NSLOT:2 * NSLOT]),
                 list(rest[2 * NSLOT:3 * NSLOT]), b_per_w=b_per_w)

    out = pl.kernel(
        body,
        out_type=jax.ShapeDtypeStruct((b, EMBED_DIM), jnp.float32),
        mesh=mesh,
        compiler_params=pltpu.CompilerParams(needs_layout_passes=False,
                                             use_tc_tiling_on_sc=False),
        scratch_types=(
            [pltpu.VMEM((b_per_w,), jnp.int32)]
            + [pltpu.VMEM((CHUNK, EMBED_DIM), jnp.float32)] * NSLOT
            + [pltpu.SemaphoreType.DMA] * (2 * NSLOT)
        ),
    )(idx, embed_mat)
    return out.reshape(b0, seq, EMBED_DIM)


# speedup vs baseline: 1.0054x; 1.0054x over previous
import jax
import jax.numpy as jnp
from jax import lax
from jax.experimental import pallas as pl
from jax.experimental.pallas import tpu as pltpu
from jax.experimental.pallas import tpu_sc as plsc

EMBED_DIM = 64
SCALE = 8.0
L = 16
NC, NS = 2, 16
NW = NC * NS
CHUNK = 256
GSIZE = 128
NSLOT = 4
NGROUP = CHUNK // GSIZE


_GATHER_DN = lax.GatherDimensionNumbers(
    offset_dims=(), collapsed_slice_dims=(0,), start_index_map=(0,))


def _splat_lane(y, k):
    idx = jnp.full((L, 1), k, jnp.int32)
    return lax.gather(y, idx, _GATHER_DN, (1,),
                      mode=lax.GatherScatterMode.PROMISE_IN_BOUNDS)


def _normalize_rows(rows_v, chunk):
    lane = lax.iota(jnp.int32, L)

    def body16(i, carry):
        r0 = i * 16
        for q in range(4):
            base = r0 + 4 * q
            acc = jnp.full((L,), 1.0, jnp.float32)
            vs = []
            for k in range(4):
                r = base + k
                v0 = rows_v[r, pl.ds(0, L)]
                v1 = rows_v[r, pl.ds(L, L)]
                v2 = rows_v[r, pl.ds(2 * L, L)]
                v3 = rows_v[r, pl.ds(3 * L, L)]
                s = jnp.sum((v0 * v0 + v1 * v1) + (v2 * v2 + v3 * v3))
                acc = jnp.where(lane == k, jnp.full((L,), s, jnp.float32), acc)
                vs.append((r, v0, v1, v2, v3))
            sv = jnp.maximum(acc, 1e-24)
            i = plsc.bitcast(sv, jnp.int32)
            y = plsc.bitcast(
                jnp.full((L,), 0x5F3759DF, jnp.int32) - (i >> 1), jnp.float32)
            for _ in range(3):
                y = y * (1.5 - 0.5 * sv * y * y)
            y = y * SCALE
            for k, (r, v0, v1, v2, v3) in enumerate(vs):
                sc = _splat_lane(y, k)
                rows_v[r, pl.ds(0, L)] = v0 * sc
                rows_v[r, pl.ds(L, L)] = v1 * sc
                rows_v[r, pl.ds(2 * L, L)] = v2 * sc
                rows_v[r, pl.ds(3 * L, L)] = v3 * sc
        return carry

    lax.fori_loop(0, chunk // 16, body16, 0)


def _sc_body(idx_hbm, table_hbm, out_hbm, idx_v, rows, gsems, osems, b_per_w):
    wid = lax.axis_index("s") * NC + lax.axis_index("c")
    base = wid * b_per_w
    nchunks = b_per_w // CHUNK

    pltpu.sync_copy(idx_hbm.at[pl.ds(base, b_per_w)], idx_v)

    def fire_gather(g, slot):
        hs = []
        for j in range(NGROUP):
            hs.append(pltpu.async_copy(
                table_hbm.at[idx_v.at[pl.ds(g * CHUNK + j * GSIZE, GSIZE)]],
                rows[slot].at[pl.ds(j * GSIZE, GSIZE)],
                gsems[slot]))
        return hs

    def wait_gather(g, slot):
        for j in range(NGROUP):
            pltpu.make_async_copy(
                table_hbm.at[idx_v.at[pl.ds(g * CHUNK + j * GSIZE, GSIZE)]],
                rows[slot].at[pl.ds(j * GSIZE, GSIZE)],
                gsems[slot]).wait()

    def fire_out(g, slot):
        return pltpu.async_copy(
            rows[slot], out_hbm.at[pl.ds(base + g * CHUNK, CHUNK)], osems[slot])

    def wait_out(g, slot):
        pltpu.make_async_copy(
            rows[slot], out_hbm.at[pl.ds(base + g * CHUNK, CHUNK)],
            osems[slot]).wait()

    fire_gather(0, 0)
    fire_gather(1, 1)

    def quad_body(g4, carry):
        for b in range(NSLOT):
            g = g4 * NSLOT + b
            nxt = (b + 2) % NSLOT

            @pl.when(g >= 2)
            def _():
                wait_out(g - 2, nxt)

            @pl.when(g + 2 < nchunks)
            def _():
                fire_gather(g + 2, nxt)

            wait_gather(g, b)
            _normalize_rows(rows[b], CHUNK)
            fire_out(g, b)
        return carry

    lax.fori_loop(0, nchunks // NSLOT, quad_body, 0)

    for g in (nchunks - 2, nchunks - 1):
        wait_out(g, g % NSLOT)


def kernel(x, embed_mat):
    b0, seq = x.shape
    b = b0 * seq
    b_per_w = b // NW
    assert b % NW == 0 and b_per_w % (NSLOT * CHUNK) == 0
    idx = x.reshape(b).astype(jnp.int32)
    mesh = plsc.VectorSubcoreMesh(core_axis_name="c", subcore_axis_name="s")
    out = pl.kernel(
        lambda idx_h, tab_h, out_h, idx_v, r0, r1, r2, r3, g0, g1, g2, g3,
               o0, o1, o2, o3: _sc_body(
            idx_h, tab_h, out_h, idx_v, [r0, r1, r2, r3],
            [g0, g1, g2, g3], [o0, o1, o2, o3], b_per_w=b_per_w),
        out_type=jax.ShapeDtypeStruct((b, EMBED_DIM), jnp.float32),
        mesh=mesh,
        compiler_params=pltpu.CompilerParams(needs_layout_passes=False,
                                             use_tc_tiling_on_sc=False),
        scratch_types=(
            [pltpu.VMEM((b_per_w,), jnp.int32)]
            + [pltpu.VMEM((CHUNK, EMBED_DIM), jnp.float32)] * NSLOT
            + [pltpu.SemaphoreType.DMA] * (2 * NSLOT)
        ),
    )(idx, embed_mat)
    return out.reshape(b0, seq, EMBED_DIM)
